# Initial kernel scaffold; baseline (speedup 1.0000x reference)
import jax
import jax.numpy as jnp
from jax.experimental import pallas as pl


def kernel(logits, targets, tissue_mask):
    raise NotImplementedError("write your pallas kernel here")


# speedup vs baseline: 12.1563x; 12.1563x over previous
import jax
import jax.numpy as jnp
from jax import lax
from jax.experimental import pallas as pl
from jax.experimental.pallas import tpu as pltpu

_KEEP_RATIO = 0.5
_B = 8
_N = 512 * 512
_LANES = 128
_ROWS = _N // _LANES
_S = 8
_CR = _ROWS // _S
_K_ALL = max(1, int(_N * _KEEP_RATIO))


def _ohem_body(x_ref, z_ref, m_ref, out_ref, nb_ref, acc_ref, res_ref):
    b = pl.program_id(0)
    s = pl.program_id(1)

    x = x_ref[0]
    z = z_ref[0]
    m = m_ref[0]

    bce = jnp.maximum(x, 0.0) - x * z + jnp.log1p(jnp.exp(-jnp.abs(x)))
    loss = bce * m
    posf = z * m
    negf = (1.0 - z) * m

    n_pos_p = jnp.sum(posf)
    s_pos_p = jnp.sum(loss * posf)
    n_neg_p = jnp.sum(negf)

    negbits = jnp.where(negf > 0.0,
                        lax.bitcast_convert_type(loss, jnp.int32),
                        jnp.int32(-1))
    nb_ref[pl.ds(s * _CR, _CR), :] = negbits

    @pl.when(jnp.logical_and(b == 0, s == 0))
    def _init_res():
        res_ref[0] = 0.0

    @pl.when(s == 0)
    def _init_acc():
        acc_ref[0] = n_pos_p
        acc_ref[1] = s_pos_p
        acc_ref[2] = n_neg_p

    @pl.when(s != 0)
    def _accum():
        acc_ref[0] += n_pos_p
        acc_ref[1] += s_pos_p
        acc_ref[2] += n_neg_p

    @pl.when(s == _S - 1)
    def _select():
        n_pos = acc_ref[0]
        sum_pos = acc_ref[1]
        num_neg = acc_ref[2]
        n_remain = jnp.maximum(0.0, jnp.float32(_K_ALL) - n_pos)
        k_eff = jnp.minimum(n_remain, num_neg)
        k_eff_i = k_eff.astype(jnp.int32)

        def count_ge(cand):
            def chunk_body(c, tot):
                blk = nb_ref[pl.ds(c * _CR, _CR), :]
                return tot + jnp.sum((blk >= cand).astype(jnp.int32))
            return lax.fori_loop(0, _S, chunk_body, jnp.int32(0))

        def bit_body(i, prefix):
            cand = prefix | lax.shift_left(jnp.int32(1), 30 - i)
            cnt = count_ge(cand)
            return jnp.where(cnt >= k_eff_i, cand, prefix)

        t_bits = lax.fori_loop(0, 31, bit_body, jnp.int32(0))
        t_val = lax.bitcast_convert_type(t_bits, jnp.float32)

        def final_body(c, carry):
            n_gt, s_gt = carry
            blk = nb_ref[pl.ds(c * _CR, _CR), :]
            gt = blk > t_bits
            vals = lax.bitcast_convert_type(blk, jnp.float32)
            n_gt = n_gt + jnp.sum(gt.astype(jnp.int32))
            s_gt = s_gt + jnp.sum(jnp.where(gt, vals, 0.0))
            return n_gt, s_gt

        n_gt, s_gt = lax.fori_loop(0, _S, final_body,
                                   (jnp.int32(0), jnp.float32(0.0)))

        sum_topk = jnp.where(
            k_eff_i > 0,
            s_gt + (k_eff - n_gt.astype(jnp.float32)) * t_val,
            jnp.float32(0.0))
        cnt = n_pos + k_eff
        img_loss = jnp.where(cnt > 0.0,
                             (sum_pos + sum_topk) / jnp.maximum(cnt, 1.0),
                             jnp.float32(0.0))
        res_ref[0] += img_loss

        @pl.when(b == _B - 1)
        def _finish():
            out_ref[0] = res_ref[0] / jnp.float32(_B)


def kernel(logits, targets, tissue_mask):
    xs = logits.reshape(_B, _ROWS, _LANES)
    zs = targets.reshape(_B, _ROWS, _LANES)
    ms = tissue_mask.reshape(_B, _ROWS, _LANES)

    in_spec = pl.BlockSpec((1, _CR, _LANES), lambda b, s: (b, s, 0))
    out = pl.pallas_call(
        _ohem_body,
        grid=(_B, _S),
        in_specs=[in_spec, in_spec, in_spec],
        out_specs=pl.BlockSpec(memory_space=pltpu.SMEM),
        out_shape=jax.ShapeDtypeStruct((1,), jnp.float32),
        scratch_shapes=[
            pltpu.VMEM((_ROWS, _LANES), jnp.int32),
            pltpu.SMEM((3,), jnp.float32),
            pltpu.SMEM((1,), jnp.float32),
        ],
    )(xs, zs, ms)
    return out[0]


# speedup vs baseline: 27.2570x; 2.2422x over previous
import jax
import jax.numpy as jnp
from jax import lax
from jax.experimental import pallas as pl
from jax.experimental.pallas import tpu as pltpu

_KEEP_RATIO = 0.5
_B = 8
_N = 512 * 512
_LANES = 128
_ROWS = _N // _LANES
_S = 8
_CR = _ROWS // _S
_K_ALL = max(1, int(_N * _KEEP_RATIO))


def _ohem_body(x_ref, z_ref, m_ref, out_ref, nb_ref, acc_ref, res_ref):
    b = pl.program_id(0)
    s = pl.program_id(1)

    x = x_ref[0]
    z = z_ref[0]
    m = m_ref[0]

    bce = jnp.maximum(x, 0.0) - x * z + jnp.log1p(jnp.exp(-jnp.abs(x)))
    loss = bce * m
    posf = z * m
    negf = (1.0 - z) * m

    n_pos_p = jnp.sum(posf)
    s_pos_p = jnp.sum(loss * posf)
    n_neg_p = jnp.sum(negf)

    negbits = jnp.where(negf > 0.0,
                        lax.bitcast_convert_type(loss, jnp.int32),
                        jnp.int32(-1))
    nb_ref[pl.ds(s * _CR, _CR), :] = negbits

    @pl.when(jnp.logical_and(b == 0, s == 0))
    def _init_res():
        res_ref[0] = 0.0

    @pl.when(s == 0)
    def _init_acc():
        acc_ref[0] = n_pos_p
        acc_ref[1] = s_pos_p
        acc_ref[2] = n_neg_p

    @pl.when(s != 0)
    def _accum():
        acc_ref[0] += n_pos_p
        acc_ref[1] += s_pos_p
        acc_ref[2] += n_neg_p

    @pl.when(s == _S - 1)
    def _select():
        n_pos = acc_ref[0]
        sum_pos = acc_ref[1]
        num_neg = acc_ref[2]
        n_remain = jnp.maximum(0.0, jnp.float32(_K_ALL) - n_pos)
        k_eff = jnp.minimum(n_remain, num_neg)
        k_eff_i = k_eff.astype(jnp.int32)

        _VR = 64

        def count_ge(cand):
            def chunk_body(c, acc):
                blk = nb_ref[pl.ds(c * _VR, _VR), :]
                return acc + (blk >= cand).astype(jnp.int32)
            acc = lax.fori_loop(0, _ROWS // _VR, chunk_body,
                                jnp.zeros((_VR, _LANES), jnp.int32))
            return jnp.sum(acc)

        def bit_body(i, prefix):
            cand = prefix | lax.shift_left(jnp.int32(1), 30 - i)
            cnt = count_ge(cand)
            return jnp.where(cnt >= k_eff_i, cand, prefix)

        t_bits = lax.fori_loop(0, 31, bit_body, jnp.int32(0))
        t_val = lax.bitcast_convert_type(t_bits, jnp.float32)

        def final_body(c, carry):
            an, av = carry
            blk = nb_ref[pl.ds(c * _VR, _VR), :]
            gt = blk > t_bits
            vals = lax.bitcast_convert_type(blk, jnp.float32)
            an = an + gt.astype(jnp.int32)
            av = av + jnp.where(gt, vals, 0.0)
            return an, av

        an, av = lax.fori_loop(0, _ROWS // _VR, final_body,
                               (jnp.zeros((_VR, _LANES), jnp.int32),
                                jnp.zeros((_VR, _LANES), jnp.float32)))
        n_gt = jnp.sum(an)
        s_gt = jnp.sum(av)

        sum_topk = jnp.where(
            k_eff_i > 0,
            s_gt + (k_eff - n_gt.astype(jnp.float32)) * t_val,
            jnp.float32(0.0))
        cnt = n_pos + k_eff
        img_loss = jnp.where(cnt > 0.0,
                             (sum_pos + sum_topk) / jnp.maximum(cnt, 1.0),
                             jnp.float32(0.0))
        res_ref[0] += img_loss

        @pl.when(b == _B - 1)
        def _finish():
            out_ref[0] = res_ref[0] / jnp.float32(_B)


def kernel(logits, targets, tissue_mask):
    xs = logits.reshape(_B, _ROWS, _LANES)
    zs = targets.reshape(_B, _ROWS, _LANES)
    ms = tissue_mask.reshape(_B, _ROWS, _LANES)

    in_spec = pl.BlockSpec((1, _CR, _LANES), lambda b, s: (b, s, 0))
    out = pl.pallas_call(
        _ohem_body,
        grid=(_B, _S),
        in_specs=[in_spec, in_spec, in_spec],
        out_specs=pl.BlockSpec(memory_space=pltpu.SMEM),
        out_shape=jax.ShapeDtypeStruct((1,), jnp.float32),
        scratch_shapes=[
            pltpu.VMEM((_ROWS, _LANES), jnp.int32),
            pltpu.SMEM((3,), jnp.float32),
            pltpu.SMEM((1,), jnp.float32),
        ],
    )(xs, zs, ms)
    return out[0]


# speedup vs baseline: 27.6167x; 1.0132x over previous
import jax
import jax.numpy as jnp
from jax import lax
from jax.experimental import pallas as pl
from jax.experimental.pallas import tpu as pltpu

_KEEP_RATIO = 0.5
_B = 8
_N = 512 * 512
_LANES = 128
_ROWS = _N // _LANES
_S = 8
_CR = _ROWS // _S
_K_ALL = max(1, int(_N * _KEEP_RATIO))


def _ohem_body(x_ref, z_ref, m_ref, out_ref, nb_ref, acc_ref, res_ref):
    b = pl.program_id(0)
    s = pl.program_id(1)

    x = x_ref[0]
    z = z_ref[0]
    m = m_ref[0]

    bce = jnp.maximum(x, 0.0) - x * z + jnp.log1p(jnp.exp(-jnp.abs(x)))
    loss = bce * m
    posf = z * m
    negf = (1.0 - z) * m

    n_pos_p = jnp.sum(posf)
    s_pos_p = jnp.sum(loss * posf)
    n_neg_p = jnp.sum(negf)

    negbits = jnp.where(negf > 0.0,
                        lax.bitcast_convert_type(loss, jnp.int32),
                        jnp.int32(-1))
    nb_ref[pl.ds(s * _CR, _CR), :] = negbits

    @pl.when(jnp.logical_and(b == 0, s == 0))
    def _init_res():
        res_ref[0] = 0.0

    @pl.when(s == 0)
    def _init_acc():
        acc_ref[0] = n_pos_p
        acc_ref[1] = s_pos_p
        acc_ref[2] = n_neg_p

    @pl.when(s != 0)
    def _accum():
        acc_ref[0] += n_pos_p
        acc_ref[1] += s_pos_p
        acc_ref[2] += n_neg_p

    @pl.when(s == _S - 1)
    def _select():
        n_pos = acc_ref[0]
        sum_pos = acc_ref[1]
        num_neg = acc_ref[2]
        n_remain = jnp.maximum(0.0, jnp.float32(_K_ALL) - n_pos)
        k_eff = jnp.minimum(n_remain, num_neg)
        k_eff_i = k_eff.astype(jnp.int32)

        _VR = 64

        def count_ge(cand):
            def chunk_body(c, acc):
                blk = nb_ref[pl.ds(c * _VR, _VR), :]
                return acc + (blk >= cand).astype(jnp.int32)
            acc = lax.fori_loop(0, _ROWS // _VR, chunk_body,
                                jnp.zeros((_VR, _LANES), jnp.int32))
            return jnp.sum(acc)

        prefix0 = jnp.where(count_ge(jnp.int32(1 << 30)) >= k_eff_i,
                            jnp.int32(1 << 30), jnp.int32(0))

        def pair_body(i, prefix):
            sh = 28 - 2 * i
            c1 = prefix | lax.shift_left(jnp.int32(1), sh)
            c2 = prefix | lax.shift_left(jnp.int32(2), sh)
            c3 = prefix | lax.shift_left(jnp.int32(3), sh)

            def chunk_body(c, accs):
                a1, a2, a3 = accs
                blk = nb_ref[pl.ds(c * _VR, _VR), :]
                a1 = a1 + (blk >= c1).astype(jnp.int32)
                a2 = a2 + (blk >= c2).astype(jnp.int32)
                a3 = a3 + (blk >= c3).astype(jnp.int32)
                return a1, a2, a3

            z = jnp.zeros((_VR, _LANES), jnp.int32)
            a1, a2, a3 = lax.fori_loop(0, _ROWS // _VR, chunk_body, (z, z, z))
            j = ((jnp.sum(a1) >= k_eff_i).astype(jnp.int32)
                 + (jnp.sum(a2) >= k_eff_i).astype(jnp.int32)
                 + (jnp.sum(a3) >= k_eff_i).astype(jnp.int32))
            return prefix | lax.shift_left(j, sh)

        t_bits = lax.fori_loop(0, 15, pair_body, prefix0)
        t_val = lax.bitcast_convert_type(t_bits, jnp.float32)

        def final_body(c, carry):
            an, av = carry
            blk = nb_ref[pl.ds(c * _VR, _VR), :]
            gt = blk > t_bits
            vals = lax.bitcast_convert_type(blk, jnp.float32)
            an = an + gt.astype(jnp.int32)
            av = av + jnp.where(gt, vals, 0.0)
            return an, av

        an, av = lax.fori_loop(0, _ROWS // _VR, final_body,
                               (jnp.zeros((_VR, _LANES), jnp.int32),
                                jnp.zeros((_VR, _LANES), jnp.float32)))
        n_gt = jnp.sum(an)
        s_gt = jnp.sum(av)

        sum_topk = jnp.where(
            k_eff_i > 0,
            s_gt + (k_eff - n_gt.astype(jnp.float32)) * t_val,
            jnp.float32(0.0))
        cnt = n_pos + k_eff
        img_loss = jnp.where(cnt > 0.0,
                             (sum_pos + sum_topk) / jnp.maximum(cnt, 1.0),
                             jnp.float32(0.0))
        res_ref[0] += img_loss

        @pl.when(b == _B - 1)
        def _finish():
            out_ref[0] = res_ref[0] / jnp.float32(_B)


def kernel(logits, targets, tissue_mask):
    xs = logits.reshape(_B, _ROWS, _LANES)
    zs = targets.reshape(_B, _ROWS, _LANES)
    ms = tissue_mask.reshape(_B, _ROWS, _LANES)

    in_spec = pl.BlockSpec((1, _CR, _LANES), lambda b, s: (b, s, 0))
    out = pl.pallas_call(
        _ohem_body,
        grid=(_B, _S),
        in_specs=[in_spec, in_spec, in_spec],
        out_specs=pl.BlockSpec(memory_space=pltpu.SMEM),
        out_shape=jax.ShapeDtypeStruct((1,), jnp.float32),
        scratch_shapes=[
            pltpu.VMEM((_ROWS, _LANES), jnp.int32),
            pltpu.SMEM((3,), jnp.float32),
            pltpu.SMEM((1,), jnp.float32),
        ],
    )(xs, zs, ms)
    return out[0]
